# baseline (speedup 1.0000x reference)
import functools

import jax
import jax.numpy as jnp
from jax import lax
from jax.experimental import pallas as pl
from jax.experimental.pallas import tpu as pltpu
from jax.experimental.pallas import tpu_sc as plsc

NC = 2
NS = 16
NW = NC * NS
CHUNK = 128
NB = 2
IDXB = 40
ROWS_FRAC0 = 0.25
DEGW = 16
RB = 1024


def _nacc(n):
    return ((n + 1 + 255) // 256) * 256


def _mesh():
    return plsc.VectorSubcoreMesh(
        core_axis_name="c", subcore_axis_name="s",
        num_cores=NC, num_subcores=NS)


def _make_deg(n, d, e_pad):
    ept = e_pad // NW
    nchunks = ept // CHUNK
    nacc = _nacc(n)
    rpt = nacc // NS
    opt = nacc // NS
    assert rpt % 16 == 0 and ept % CHUNK == 0

    @functools.partial(
        pl.kernel,
        out_type=jax.ShapeDtypeStruct((NC, nacc, d), jnp.float32),
        mesh=_mesh(),
        scratch_types=[
            pltpu.VMEM((nchunks, CHUNK), jnp.int32),
            pltpu.VMEM((CHUNK, d), jnp.float32),
            pltpu.VMEM((16, d), jnp.float32),
            pltpu.VMEM_SHARED((nacc, d), jnp.float32),
        ],
    )
    def deg_kernel(dst2d, out, didx2d, ones, z16, acc):
        c = lax.axis_index("c")
        s = lax.axis_index("s")
        tile = c * NS + s

        pltpu.sync_copy(dst2d.at[pl.ds(tile * nchunks, nchunks)], didx2d)

        zv = jnp.zeros((16,), jnp.float32)
        ov = jnp.ones((16,), jnp.float32)
        for i in range(16):
            for j in range(d // 16):
                z16[i, pl.ds(j * 16, 16)] = zv
        def ones_body(i, _):
            for j in range(d // 16):
                ones[i, pl.ds(j * 16, 16)] = ov
            return 0
        lax.fori_loop(0, CHUNK, ones_body, 0)

        def zero_body(i, _):
            pltpu.sync_copy(z16, acc.at[pl.ds(s * rpt + i * 16, 16)])
            return 0
        lax.fori_loop(0, rpt // 16, zero_body, 0)
        plsc.subcore_barrier()

        def body(j, _):
            pltpu.sync_copy(ones, acc.at[didx2d.at[j]], add=True)
            return 0
        lax.fori_loop(0, nchunks, body, 0)
        plsc.subcore_barrier()

        pltpu.sync_copy(acc.at[pl.ds(s * opt, opt)],
                        out.at[c, pl.ds(s * opt, opt)])

    return deg_kernel


def _split_chunks(total_chunks, frac0):
    unit = NS * 8
    ch0 = int(round(total_chunks * frac0 / unit)) * unit
    ch0 = max(unit, min(total_chunks - unit, ch0))
    return ch0 // NS, (total_chunks - ch0) // NS


def _blocks(nch):
    out, off = [], 0
    while off < nch:
        b = min(IDXB, nch - off)
        out.append((off, b))
        off += b
    return out


def _make_row_scatter(n, d, e_pad, frac0):
    total_chunks = e_pad // CHUNK
    nacc = _nacc(n)
    rpt = nacc // NS
    opt = nacc // NS
    nch0, nch1 = _split_chunks(total_chunks, frac0)
    assert rpt % 16 == 0 and NS * (nch0 + nch1) == total_chunks

    @functools.partial(
        pl.kernel,
        out_type=jax.ShapeDtypeStruct((NC, nacc, d), jnp.float32),
        mesh=_mesh(),
        scratch_types=[
            pltpu.VMEM((IDXB, CHUNK), jnp.int32),
            pltpu.VMEM((IDXB, CHUNK), jnp.int32),
            [pltpu.VMEM((CHUNK, d), jnp.float32) for _ in range(NB)],
            pltpu.VMEM((8, d), jnp.float32),
            pltpu.VMEM_SHARED((nacc, d), jnp.float32),
            [pltpu.SemaphoreType.DMA for _ in range(NB)],
        ],
    )
    def scatter_rows(hs, src2d, dst2d, out, sidx2d, didx2d, rows, z8, acc,
                     semg):
        c = lax.axis_index("c")
        s = lax.axis_index("s")

        zv = jnp.zeros((16,), jnp.float32)
        for i in range(8):
            for j in range(d // 16):
                z8[i, pl.ds(j * 16, 16)] = zv

        def zero_body(i, _):
            pltpu.sync_copy(z8, acc.at[pl.ds(s * rpt + i * 8, 8)])
            return 0
        lax.fori_loop(0, rpt // 8, zero_body, 0)
        plsc.subcore_barrier()

        def run_core(nch, core_start):
            first = core_start + s * nch
            for blk_off, blk_len in _blocks(nch):
                bbase = first + blk_off
                pltpu.sync_copy(src2d.at[pl.ds(bbase, blk_len)],
                                sidx2d.at[pl.ds(0, blk_len)])
                pltpu.sync_copy(dst2d.at[pl.ds(bbase, blk_len)],
                                didx2d.at[pl.ds(0, blk_len)])
                for b in range(NB):
                    pltpu.async_copy(hs.at[sidx2d.at[b]], rows[b], semg[b])

                def group_body(i, _):
                    for b in range(NB):
                        j = i * NB + b
                        pltpu.make_async_copy(hs.at[sidx2d.at[j]], rows[b],
                                              semg[b]).wait()
                        pltpu.sync_copy(rows[b], acc.at[didx2d.at[j]],
                                        add=True)
                        @pl.when(j + NB < blk_len)
                        def _():
                            pltpu.async_copy(hs.at[sidx2d.at[j + NB]],
                                             rows[b], semg[b])
                    return 0
                lax.fori_loop(0, blk_len // NB, group_body, 0)

        @pl.when(c == 0)
        def _():
            run_core(nch0, 0)

        @pl.when(c == 1)
        def _():
            run_core(nch1, NS * nch0)

        plsc.subcore_barrier()

        pltpu.sync_copy(acc.at[pl.ds(s * opt, opt)],
                        out.at[c, pl.ds(s * opt, opt)])

    return scatter_rows


def _mm_scale(x, w, degp):
    n, d = x.shape
    grid = (pl.cdiv(n, RB),)

    def body(x_ref, w_ref, dp_ref, o_ref):
        dinv = lax.rsqrt(dp_ref[0] + dp_ref[1] + 1.0)
        o_ref[...] = jnp.dot(x_ref[...], w_ref[...],
                             preferred_element_type=jnp.float32) * dinv[:, None]

    return pl.pallas_call(
        body,
        grid=grid,
        in_specs=[
            pl.BlockSpec((RB, d), lambda i: (i, 0)),
            pl.BlockSpec((d, d), lambda i: (0, 0)),
            pl.BlockSpec((2, RB), lambda i: (0, i)),
        ],
        out_specs=pl.BlockSpec((RB, d), lambda i: (i, 0)),
        out_shape=jax.ShapeDtypeStruct((n, d), jnp.float32),
    )(x, w, degp)


def _combine_mm(aggp, hs1, degp1, degp2, w, b):
    n, d = hs1.shape
    grid = (pl.cdiv(n, RB),)

    def body(a_ref, hs_ref, d1_ref, d2_ref, w_ref, b_ref, o_ref):
        dinv1 = lax.rsqrt(d1_ref[0] + d1_ref[1] + 1.0)
        dinv2 = lax.rsqrt(d2_ref[0] + d2_ref[1] + 1.0)
        xl = jnp.maximum(
            (a_ref[0] + a_ref[1] + hs_ref[...]) * dinv1[:, None] + b_ref[...],
            0.0)
        o_ref[...] = jnp.dot(xl, w_ref[...],
                             preferred_element_type=jnp.float32) * dinv2[:, None]

    return pl.pallas_call(
        body,
        grid=grid,
        in_specs=[
            pl.BlockSpec((2, RB, d), lambda i: (0, i, 0)),
            pl.BlockSpec((RB, d), lambda i: (i, 0)),
            pl.BlockSpec((2, RB), lambda i: (0, i)),
            pl.BlockSpec((2, RB), lambda i: (0, i)),
            pl.BlockSpec((d, d), lambda i: (0, 0)),
            pl.BlockSpec((1, d), lambda i: (0, 0)),
        ],
        out_specs=pl.BlockSpec((RB, d), lambda i: (i, 0)),
        out_shape=jax.ShapeDtypeStruct((n, d), jnp.float32),
    )(aggp, hs1, degp1, degp2, w, b)


def _combine_bn(aggp, hs2, degp2, b, gamma, beta):
    n, d = hs2.shape

    def body(a_ref, hs_ref, d2_ref, b_ref, g_ref, be_ref, o_ref):
        dinv2 = lax.rsqrt(d2_ref[0, :n] + d2_ref[1, :n] + 1.0)
        xlg = jnp.maximum(
            (a_ref[0, :n] + a_ref[1, :n] + hs_ref[...]) * dinv2[:, None]
            + b_ref[...],
            0.0)
        mean = jnp.mean(xlg, axis=0)
        var = jnp.mean(xlg * xlg, axis=0) - mean * mean
        o_ref[...] = ((xlg - mean) * lax.rsqrt(var + 1e-5) * g_ref[...]
                      + be_ref[...])

    return pl.pallas_call(
        body,
        out_shape=jax.ShapeDtypeStruct((n, d), jnp.float32),
    )(aggp, hs2, degp2, b, gamma, beta)


def kernel(x, edge_index, subgraph_edge_index, W_local, b_local, W_glob,
           b_glob, gamma, beta):
    n, d = x.shape

    nacc = _nacc(n)

    def pad_edges(ei):
        e = ei.shape[1]
        e_pad = pl.cdiv(e, NW * CHUNK * IDXB) * (NW * CHUNK * IDXB)
        p = e_pad - e
        src = jnp.concatenate([ei[0], jnp.zeros((p,), ei.dtype)])
        trash = n + jnp.arange(p, dtype=ei.dtype) % (nacc - n)
        dst = jnp.concatenate([ei[1], trash])
        return (src.reshape(e_pad // CHUNK, CHUNK),
                dst.reshape(e_pad // CHUNK, CHUNK), e_pad)

    src_s, dst_s, ep_s = pad_edges(subgraph_edge_index)
    src_g, dst_g, ep_g = pad_edges(edge_index)

    degp_s16 = _make_deg(n, d, ep_s)(dst_s)
    degp_g16 = _make_deg(n, d, ep_g)(dst_g)
    degp_s = degp_s16[:, :, 0]
    degp_g = degp_g16[:, :, 0]

    hs1 = _mm_scale(x, W_local, degp_s)
    agg1 = _make_row_scatter(n, d, ep_s, ROWS_FRAC0)(hs1, src_s, dst_s)
    hs2 = _combine_mm(agg1, hs1, degp_s, degp_g, W_glob,
                      b_local.reshape(1, d))
    agg2 = _make_row_scatter(n, d, ep_g, ROWS_FRAC0)(hs2, src_g, dst_g)
    return _combine_bn(agg2, hs2, degp_g, b_glob.reshape(1, d),
                       gamma.reshape(1, d), beta.reshape(1, d))


# speedup vs baseline: 1.0825x; 1.0825x over previous
import functools

import jax
import jax.numpy as jnp
from jax import lax
from jax.experimental import pallas as pl
from jax.experimental.pallas import tpu as pltpu
from jax.experimental.pallas import tpu_sc as plsc

NC = 2
NS = 16
NW = NC * NS
CHUNK = 128
NB = 2
IDXB = 40
ROWS_FRAC0 = 0.75
DEGW = 16
RB = 1024


def _nacc(n):
    return ((n + 1 + 255) // 256) * 256


def _mesh():
    return plsc.VectorSubcoreMesh(
        core_axis_name="c", subcore_axis_name="s",
        num_cores=NC, num_subcores=NS)


def _make_deg(n, d, e_pad):
    ept = e_pad // NW
    nchunks = ept // CHUNK
    nacc = _nacc(n)
    rpt = nacc // NS
    opt = nacc // NS
    assert rpt % 16 == 0 and ept % CHUNK == 0

    @functools.partial(
        pl.kernel,
        out_type=jax.ShapeDtypeStruct((NC, nacc, d), jnp.float32),
        mesh=_mesh(),
        scratch_types=[
            pltpu.VMEM((nchunks, CHUNK), jnp.int32),
            pltpu.VMEM((CHUNK, d), jnp.float32),
            pltpu.VMEM((16, d), jnp.float32),
            pltpu.VMEM_SHARED((nacc, d), jnp.float32),
        ],
    )
    def deg_kernel(dst2d, out, didx2d, ones, z16, acc):
        c = lax.axis_index("c")
        s = lax.axis_index("s")
        tile = c * NS + s

        pltpu.sync_copy(dst2d.at[pl.ds(tile * nchunks, nchunks)], didx2d)

        zv = jnp.zeros((16,), jnp.float32)
        ov = jnp.ones((16,), jnp.float32)
        for i in range(16):
            for j in range(d // 16):
                z16[i, pl.ds(j * 16, 16)] = zv
        def ones_body(i, _):
            for j in range(d // 16):
                ones[i, pl.ds(j * 16, 16)] = ov
            return 0
        lax.fori_loop(0, CHUNK, ones_body, 0)

        def zero_body(i, _):
            pltpu.sync_copy(z16, acc.at[pl.ds(s * rpt + i * 16, 16)])
            return 0
        lax.fori_loop(0, rpt // 16, zero_body, 0)
        plsc.subcore_barrier()

        def body(j, _):
            pltpu.sync_copy(ones, acc.at[didx2d.at[j]], add=True)
            return 0
        lax.fori_loop(0, nchunks, body, 0)
        plsc.subcore_barrier()

        pltpu.sync_copy(acc.at[pl.ds(s * opt, opt)],
                        out.at[c, pl.ds(s * opt, opt)])

    return deg_kernel


def _split_chunks(total_chunks, frac0):
    unit = NS * 8
    ch0 = int(round(total_chunks * frac0 / unit)) * unit
    ch0 = max(unit, min(total_chunks - unit, ch0))
    return ch0 // NS, (total_chunks - ch0) // NS


def _blocks(nch):
    out, off = [], 0
    while off < nch:
        b = min(IDXB, nch - off)
        out.append((off, b))
        off += b
    return out


def _make_row_scatter(n, d, e_pad, frac0):
    total_chunks = e_pad // CHUNK
    nacc = _nacc(n)
    rpt = nacc // NS
    opt = nacc // NS
    nch0, nch1 = _split_chunks(total_chunks, frac0)
    assert rpt % 16 == 0 and NS * (nch0 + nch1) == total_chunks

    @functools.partial(
        pl.kernel,
        out_type=jax.ShapeDtypeStruct((NC, nacc, d), jnp.float32),
        mesh=_mesh(),
        scratch_types=[
            pltpu.VMEM((IDXB, CHUNK), jnp.int32),
            pltpu.VMEM((IDXB, CHUNK), jnp.int32),
            [pltpu.VMEM((CHUNK, d), jnp.float32) for _ in range(NB)],
            pltpu.VMEM((8, d), jnp.float32),
            pltpu.VMEM_SHARED((nacc, d), jnp.float32),
            [pltpu.SemaphoreType.DMA for _ in range(NB)],
        ],
    )
    def scatter_rows(hs, src2d, dst2d, out, sidx2d, didx2d, rows, z8, acc,
                     semg):
        c = lax.axis_index("c")
        s = lax.axis_index("s")

        zv = jnp.zeros((16,), jnp.float32)
        for i in range(8):
            for j in range(d // 16):
                z8[i, pl.ds(j * 16, 16)] = zv

        def zero_body(i, _):
            pltpu.sync_copy(z8, acc.at[pl.ds(s * rpt + i * 8, 8)])
            return 0
        lax.fori_loop(0, rpt // 8, zero_body, 0)
        plsc.subcore_barrier()

        def run_core(nch, core_start):
            first = core_start + s * nch
            for blk_off, blk_len in _blocks(nch):
                bbase = first + blk_off
                pltpu.sync_copy(src2d.at[pl.ds(bbase, blk_len)],
                                sidx2d.at[pl.ds(0, blk_len)])
                pltpu.sync_copy(dst2d.at[pl.ds(bbase, blk_len)],
                                didx2d.at[pl.ds(0, blk_len)])
                for b in range(NB):
                    pltpu.async_copy(hs.at[sidx2d.at[b]], rows[b], semg[b])

                def group_body(i, _):
                    for b in range(NB):
                        j = i * NB + b
                        pltpu.make_async_copy(hs.at[sidx2d.at[j]], rows[b],
                                              semg[b]).wait()
                        pltpu.sync_copy(rows[b], acc.at[didx2d.at[j]],
                                        add=True)
                        @pl.when(j + NB < blk_len)
                        def _():
                            pltpu.async_copy(hs.at[sidx2d.at[j + NB]],
                                             rows[b], semg[b])
                    return 0
                lax.fori_loop(0, blk_len // NB, group_body, 0)

        @pl.when(c == 0)
        def _():
            run_core(nch0, 0)

        @pl.when(c == 1)
        def _():
            run_core(nch1, NS * nch0)

        plsc.subcore_barrier()

        pltpu.sync_copy(acc.at[pl.ds(s * opt, opt)],
                        out.at[c, pl.ds(s * opt, opt)])

    return scatter_rows


def _mm_scale(x, w, degp):
    n, d = x.shape
    grid = (pl.cdiv(n, RB),)

    def body(x_ref, w_ref, dp_ref, o_ref):
        dinv = lax.rsqrt(dp_ref[0] + dp_ref[1] + 1.0)
        o_ref[...] = jnp.dot(x_ref[...], w_ref[...],
                             preferred_element_type=jnp.float32) * dinv[:, None]

    return pl.pallas_call(
        body,
        grid=grid,
        in_specs=[
            pl.BlockSpec((RB, d), lambda i: (i, 0)),
            pl.BlockSpec((d, d), lambda i: (0, 0)),
            pl.BlockSpec((2, RB), lambda i: (0, i)),
        ],
        out_specs=pl.BlockSpec((RB, d), lambda i: (i, 0)),
        out_shape=jax.ShapeDtypeStruct((n, d), jnp.float32),
    )(x, w, degp)


def _combine_mm(aggp, hs1, degp1, degp2, w, b):
    n, d = hs1.shape
    grid = (pl.cdiv(n, RB),)

    def body(a_ref, hs_ref, d1_ref, d2_ref, w_ref, b_ref, o_ref):
        dinv1 = lax.rsqrt(d1_ref[0] + d1_ref[1] + 1.0)
        dinv2 = lax.rsqrt(d2_ref[0] + d2_ref[1] + 1.0)
        xl = jnp.maximum(
            (a_ref[0] + a_ref[1] + hs_ref[...]) * dinv1[:, None] + b_ref[...],
            0.0)
        o_ref[...] = jnp.dot(xl, w_ref[...],
                             preferred_element_type=jnp.float32) * dinv2[:, None]

    return pl.pallas_call(
        body,
        grid=grid,
        in_specs=[
            pl.BlockSpec((2, RB, d), lambda i: (0, i, 0)),
            pl.BlockSpec((RB, d), lambda i: (i, 0)),
            pl.BlockSpec((2, RB), lambda i: (0, i)),
            pl.BlockSpec((2, RB), lambda i: (0, i)),
            pl.BlockSpec((d, d), lambda i: (0, 0)),
            pl.BlockSpec((1, d), lambda i: (0, 0)),
        ],
        out_specs=pl.BlockSpec((RB, d), lambda i: (i, 0)),
        out_shape=jax.ShapeDtypeStruct((n, d), jnp.float32),
    )(aggp, hs1, degp1, degp2, w, b)


def _combine_bn(aggp, hs2, degp2, b, gamma, beta):
    n, d = hs2.shape

    def body(a_ref, hs_ref, d2_ref, b_ref, g_ref, be_ref, o_ref):
        dinv2 = lax.rsqrt(d2_ref[0, :n] + d2_ref[1, :n] + 1.0)
        xlg = jnp.maximum(
            (a_ref[0, :n] + a_ref[1, :n] + hs_ref[...]) * dinv2[:, None]
            + b_ref[...],
            0.0)
        mean = jnp.mean(xlg, axis=0)
        var = jnp.mean(xlg * xlg, axis=0) - mean * mean
        o_ref[...] = ((xlg - mean) * lax.rsqrt(var + 1e-5) * g_ref[...]
                      + be_ref[...])

    return pl.pallas_call(
        body,
        out_shape=jax.ShapeDtypeStruct((n, d), jnp.float32),
    )(aggp, hs2, degp2, b, gamma, beta)


def kernel(x, edge_index, subgraph_edge_index, W_local, b_local, W_glob,
           b_glob, gamma, beta):
    n, d = x.shape

    nacc = _nacc(n)

    def pad_edges(ei):
        e = ei.shape[1]
        e_pad = pl.cdiv(e, NW * CHUNK * IDXB) * (NW * CHUNK * IDXB)
        p = e_pad - e
        src = jnp.concatenate([ei[0], jnp.zeros((p,), ei.dtype)])
        trash = n + jnp.arange(p, dtype=ei.dtype) % (nacc - n)
        dst = jnp.concatenate([ei[1], trash])
        return (src.reshape(e_pad // CHUNK, CHUNK),
                dst.reshape(e_pad // CHUNK, CHUNK), e_pad)

    src_s, dst_s, ep_s = pad_edges(subgraph_edge_index)
    src_g, dst_g, ep_g = pad_edges(edge_index)

    degp_s16 = _make_deg(n, d, ep_s)(dst_s)
    degp_g16 = _make_deg(n, d, ep_g)(dst_g)
    degp_s = degp_s16[:, :, 0]
    degp_g = degp_g16[:, :, 0]

    hs1 = _mm_scale(x, W_local, degp_s)
    agg1 = _make_row_scatter(n, d, ep_s, ROWS_FRAC0)(hs1, src_s, dst_s)
    hs2 = _combine_mm(agg1, hs1, degp_s, degp_g, W_glob,
                      b_local.reshape(1, d))
    agg2 = _make_row_scatter(n, d, ep_g, ROWS_FRAC0)(hs2, src_g, dst_g)
    return _combine_bn(agg2, hs2, degp_g, b_glob.reshape(1, d),
                       gamma.reshape(1, d), beta.reshape(1, d))
